# baseline (speedup 1.0000x reference)
import functools

import jax
import jax.numpy as jnp
from jax import lax
from jax.experimental import pallas as pl
from jax.experimental.pallas import tpu as pltpu
from jax.experimental.pallas import tpu_sc as plsc

NC, NS, LANES = 2, 16, 16
NW = NC * NS

V = 100000
D = 64
B = 4096
L = 200

C1 = 80
NCH1 = V // C1
ITER1 = (NCH1 + NW - 1) // NW

BPW = B // NW
TPW = BPW * L
LG0 = 128
LG1 = L - LG0

_MESH = plsc.VectorSubcoreMesh(
    core_axis_name="c", subcore_axis_name="s", num_cores=NC, num_subcores=NS
)
_LINEAR = pltpu.CompilerParams(use_tc_tiling_on_sc=False)


def _wid():
  return lax.axis_index("s") * NC + lax.axis_index("c")


@functools.partial(
    pl.kernel,
    out_type=jax.ShapeDtypeStruct((V, D), jnp.float32),
    mesh=_MESH,
    compiler_params=_LINEAR,
    scratch_types=[
        pltpu.VMEM((2, C1), jnp.int32),
        pltpu.VMEM((2, C1), jnp.int32),
        pltpu.VMEM((2, C1, D), jnp.float32),
        pltpu.VMEM((2, C1, D), jnp.float32),
        pltpu.VMEM((2, C1, D), jnp.float32),
        pltpu.SemaphoreType.DMA,
        pltpu.SemaphoreType.DMA,
    ],
)
def _build_t2(table, pmap, qmap, t2, pidx, qidx, wrows, prows, qrows, s0, s1):
  wid = _wid()
  sems = (s0, s1)

  def fetch(i, slot):
    base = i * C1
    sem = sems[slot]
    pltpu.sync_copy(pmap.at[pl.ds(base, C1)], pidx.at[slot])
    pltpu.sync_copy(qmap.at[pl.ds(base, C1)], qidx.at[slot])
    pltpu.async_copy(table.at[pl.ds(base, C1)], wrows.at[slot], sem)
    pltpu.async_copy(table.at[pidx.at[slot]], prows.at[slot], sem)
    pltpu.async_copy(table.at[qidx.at[slot]], qrows.at[slot], sem)

  def drain(i, slot):
    sem = sems[slot]
    pltpu.make_async_copy(table.at[pl.ds(0, C1)], wrows.at[slot], sem).wait()
    pltpu.make_async_copy(table.at[pl.ds(0, C1)], prows.at[slot], sem).wait()
    pltpu.make_async_copy(table.at[pl.ds(0, C1)], qrows.at[slot], sem).wait()

    def row(r, carry):
      for rr in range(2):
        for j in range(D // LANES):
          s = pl.ds(j * LANES, LANES)
          wrows[slot, 2 * r + rr, s] = (
              wrows[slot, 2 * r + rr, s]
              + prows[slot, 2 * r + rr, s]
              + qrows[slot, 2 * r + rr, s]
          )
      return carry

    lax.fori_loop(0, C1 // 2, row, 0)
    pltpu.sync_copy(wrows.at[slot], t2.at[pl.ds(i * C1, C1)])

  fetch(wid, 0)

  def body(m, carry):
    ia = wid + (2 * m) * NW
    ib = ia + NW
    ic = ib + NW

    @pl.when(ib < NCH1)
    def _():
      fetch(ib, 1)

    @pl.when(ia < NCH1)
    def _():
      drain(ia, 0)

    @pl.when(ic < NCH1)
    def _():
      fetch(ic, 0)

    @pl.when(ib < NCH1)
    def _():
      drain(ib, 1)

    return carry

  lax.fori_loop(0, ITER1 // 2, body, 0)


@functools.partial(
    pl.kernel,
    out_type=jax.ShapeDtypeStruct((B, L, D), jnp.float32),
    mesh=_MESH,
    compiler_params=_LINEAR,
    scratch_types=[
        pltpu.VMEM((TPW,), jnp.int32),
        pltpu.VMEM((2, L, D), jnp.float32),
        pltpu.SemaphoreType.DMA,
        pltpu.SemaphoreType.DMA,
    ],
)
def _lookup(t2, seq, out, tok, rows, s0, s1):
  wid = _wid()
  b0 = wid * BPW
  sems = (s0, s1)
  pltpu.sync_copy(seq.at[pl.ds(b0 * L, TPW)], tok)

  def fetch(k, slot):
    sem = sems[slot]
    pltpu.async_copy(t2.at[tok.at[pl.ds(k * L, LG0)]],
                     rows.at[slot, pl.ds(0, LG0)], sem)
    pltpu.async_copy(t2.at[tok.at[pl.ds(k * L + LG0, LG1)]],
                     rows.at[slot, pl.ds(LG0, LG1)], sem)

  def drain(k, slot):
    sem = sems[slot]
    pltpu.make_async_copy(t2.at[pl.ds(0, LG0)],
                          rows.at[slot, pl.ds(0, LG0)], sem).wait()
    pltpu.make_async_copy(t2.at[pl.ds(0, LG1)],
                          rows.at[slot, pl.ds(LG0, LG1)], sem).wait()
    pltpu.sync_copy(rows.at[slot], out.at[b0 + k])

  fetch(0, 0)

  def body(m, carry):
    k = 2 * m
    fetch(k + 1, 1)
    drain(k, 0)

    @pl.when(k + 2 < BPW)
    def _():
      fetch(k + 2, 0)
    drain(k + 1, 1)
    return carry

  lax.fori_loop(0, BPW // 2, body, 0)


@jax.jit
def kernel(sequence, table, prefix_map, postfix_map):
  t2 = _build_t2(table, prefix_map, postfix_map)
  return _lookup(t2, sequence.reshape(-1))


# speedup vs baseline: 1.0299x; 1.0299x over previous
import functools

import jax
import jax.numpy as jnp
from jax import lax
from jax.experimental import pallas as pl
from jax.experimental.pallas import tpu as pltpu
from jax.experimental.pallas import tpu_sc as plsc

NC, NS, LANES = 2, 16, 16
NW = NC * NS

V = 100000
D = 64
B = 4096
L = 200

C1 = 80
NCH1 = V // C1
ITER1 = (NCH1 + NW - 1) // NW

BPW = B // NW
TPW = BPW * L
LG0 = 128
LG1 = L - LG0

_MESH = plsc.VectorSubcoreMesh(
    core_axis_name="c", subcore_axis_name="s", num_cores=NC, num_subcores=NS
)
_LINEAR = pltpu.CompilerParams(use_tc_tiling_on_sc=False)


def _wid():
  return lax.axis_index("s") * NC + lax.axis_index("c")


@functools.partial(
    pl.kernel,
    out_type=jax.ShapeDtypeStruct((V, D), jnp.float32),
    mesh=_MESH,
    compiler_params=_LINEAR,
    scratch_types=[
        pltpu.VMEM((2, C1), jnp.int32),
        pltpu.VMEM((2, C1), jnp.int32),
        pltpu.VMEM((2, C1, D), jnp.float32),
        pltpu.VMEM((2, C1, D), jnp.float32),
        pltpu.VMEM((2, C1, D), jnp.float32),
        pltpu.SemaphoreType.DMA,
        pltpu.SemaphoreType.DMA,
    ],
)
def _build_t2(table, pmap, qmap, t2, pidx, qidx, wrows, prows, qrows, s0, s1):
  wid = _wid()
  sems = (s0, s1)

  def fetch(i, slot):
    base = i * C1
    sem = sems[slot]
    pltpu.sync_copy(pmap.at[pl.ds(base, C1)], pidx.at[slot])
    pltpu.sync_copy(qmap.at[pl.ds(base, C1)], qidx.at[slot])
    pltpu.async_copy(table.at[pl.ds(base, C1)], wrows.at[slot], sem)
    pltpu.async_copy(table.at[pidx.at[slot]], prows.at[slot], sem)
    pltpu.async_copy(table.at[qidx.at[slot]], qrows.at[slot], sem)

  def drain(i, slot):
    sem = sems[slot]
    pltpu.make_async_copy(table.at[pl.ds(0, C1)], wrows.at[slot], sem).wait()
    pltpu.make_async_copy(table.at[pl.ds(0, C1)], prows.at[slot], sem).wait()
    pltpu.make_async_copy(table.at[pl.ds(0, C1)], qrows.at[slot], sem).wait()

    def row(r, carry):
      for rr in range(2):
        for j in range(D // LANES):
          s = pl.ds(j * LANES, LANES)
          wrows[slot, 2 * r + rr, s] = (
              wrows[slot, 2 * r + rr, s]
              + prows[slot, 2 * r + rr, s]
              + qrows[slot, 2 * r + rr, s]
          )
      return carry

    lax.fori_loop(0, C1 // 2, row, 0)
    pltpu.sync_copy(wrows.at[slot], t2.at[pl.ds(i * C1, C1)])

  fetch(wid, 0)

  def body(m, carry):
    ia = wid + (2 * m) * NW
    ib = ia + NW
    ic = ib + NW

    @pl.when(ib < NCH1)
    def _():
      fetch(ib, 1)

    @pl.when(ia < NCH1)
    def _():
      drain(ia, 0)

    @pl.when(ic < NCH1)
    def _():
      fetch(ic, 0)

    @pl.when(ib < NCH1)
    def _():
      drain(ib, 1)

    return carry

  lax.fori_loop(0, ITER1 // 2, body, 0)


@functools.partial(
    pl.kernel,
    out_type=jax.ShapeDtypeStruct((B, L, D), jnp.float32),
    mesh=_MESH,
    compiler_params=_LINEAR,
    scratch_types=[
        pltpu.VMEM((TPW,), jnp.int32),
        pltpu.VMEM((4, L, D), jnp.float32),
        [pltpu.SemaphoreType.DMA] * 4,
        [pltpu.SemaphoreType.DMA] * 4,
    ],
)
def _lookup(t2, seq, out, tok, rows, gsems, wsems):
  wid = _wid()
  b0 = wid * BPW
  pltpu.sync_copy(seq.at[pl.ds(b0 * L, TPW)], tok)

  def fetch(k, slot):
    sem = gsems[slot]
    pltpu.async_copy(t2.at[tok.at[pl.ds(k * L, LG0)]],
                     rows.at[slot, pl.ds(0, LG0)], sem)
    pltpu.async_copy(t2.at[tok.at[pl.ds(k * L + LG0, LG1)]],
                     rows.at[slot, pl.ds(LG0, LG1)], sem)

  def wait_write(slot):
    pltpu.make_async_copy(rows.at[slot], out.at[b0], wsems[slot]).wait()

  def drain(k, slot):
    sem = gsems[slot]
    pltpu.make_async_copy(t2.at[pl.ds(0, LG0)],
                          rows.at[slot, pl.ds(0, LG0)], sem).wait()
    pltpu.make_async_copy(t2.at[pl.ds(0, LG1)],
                          rows.at[slot, pl.ds(LG0, LG1)], sem).wait()
    pltpu.async_copy(rows.at[slot], out.at[b0 + k], wsems[slot])

  fetch(0, 0)
  fetch(1, 1)

  def body(m, carry):
    k = 4 * m

    @pl.when(k >= 4)
    def _():
      wait_write(2)
    fetch(k + 2, 2)
    drain(k, 0)

    @pl.when(k >= 4)
    def _():
      wait_write(3)
    fetch(k + 3, 3)
    drain(k + 1, 1)

    @pl.when(k + 4 < BPW)
    def _():
      wait_write(0)
      fetch(k + 4, 0)
    drain(k + 2, 2)

    @pl.when(k + 5 < BPW)
    def _():
      wait_write(1)
      fetch(k + 5, 1)
    drain(k + 3, 3)
    return carry

  lax.fori_loop(0, BPW // 4, body, 0)
  for slot in range(4):
    wait_write(slot)


@jax.jit
def kernel(sequence, table, prefix_map, postfix_map):
  t2 = _build_t2(table, prefix_map, postfix_map)
  return _lookup(t2, sequence.reshape(-1))
